# Initial kernel scaffold; baseline (speedup 1.0000x reference)
import jax
import jax.numpy as jnp
from jax.experimental import pallas as pl


def kernel(x, gate_w, w1, w3, w2, s1, s3, s2):
    raise NotImplementedError("write your pallas kernel here")


# speedup vs baseline: 2.1488x; 2.1488x over previous
import functools

import jax
import jax.numpy as jnp
from jax.experimental import pallas as pl
from jax.experimental.pallas import tpu as pltpu

T = 2048
DIM = 4096
E = 64
INTER = 1024
N_GROUPS = 8
GSIZE = E // N_GROUPS
TOPK_GROUPS = 4
TOPK = 6
N_SHARED = 2
ROUTE_SCALE = 2.5

SLOTS = 8
FLAT = T * SLOTS
ROW_TILE = 128
MAX_TILES = (T * TOPK) // ROW_TILE + E
M_PAD = MAX_TILES * ROW_TILE
NEG = -1e30

GATE_TT = 256


def _gate_body(x_ref, gw_ref, idx_ref, w_ref):
    xb = x_ref[...]
    scores = jax.lax.dot_general(xb, gw_ref[...], (((1,), (1,)), ((), ())),
                                 preferred_element_type=jnp.float32)
    scores = jax.nn.sigmoid(scores)

    gcols = [jnp.max(scores[:, j * GSIZE:(j + 1) * GSIZE], axis=1, keepdims=True)
             for j in range(N_GROUPS)]
    gs = jnp.concatenate(gcols, axis=1)

    iota_g = jax.lax.broadcasted_iota(jnp.int32, (GATE_TT, N_GROUPS), 1)
    keep_g = jnp.zeros((GATE_TT, N_GROUPS), dtype=jnp.float32)
    gwork = gs
    for _ in range(TOPK_GROUPS):
        m = jnp.max(gwork, axis=1, keepdims=True)
        cand = jnp.where(gwork == m, iota_g, N_GROUPS + 1)
        am = jnp.min(cand, axis=1, keepdims=True)
        oh = iota_g == am
        keep_g = jnp.where(oh, 1.0, keep_g)
        gwork = jnp.where(oh, NEG, gwork)

    keep64 = jnp.concatenate(
        [jnp.broadcast_to(keep_g[:, j:j + 1], (GATE_TT, GSIZE))
         for j in range(N_GROUPS)], axis=1)

    ms = jnp.where(keep64 > 0.0, scores, NEG)
    iota_e = jax.lax.broadcasted_iota(jnp.int32, (GATE_TT, E), 1)
    idx_cols, val_cols = [], []
    for _ in range(TOPK):
        m = jnp.max(ms, axis=1, keepdims=True)
        cand = jnp.where(ms == m, iota_e, E + 1)
        am = jnp.min(cand, axis=1, keepdims=True)
        idx_cols.append(am)
        val_cols.append(m)
        ms = jnp.where(iota_e == am, NEG, ms)

    vals = jnp.concatenate(val_cols, axis=1)
    wsum = jnp.sum(vals, axis=1, keepdims=True)
    wnorm = vals * (ROUTE_SCALE / wsum)

    sentinel = jnp.full((GATE_TT, SLOTS - TOPK), E, dtype=jnp.int32)
    idx_ref[...] = jnp.concatenate(idx_cols + [sentinel], axis=1)
    zpad = jnp.zeros((GATE_TT, SLOTS - TOPK), dtype=jnp.float32)
    w_ref[...] = jnp.concatenate([wnorm, zpad], axis=1)


def _gate(x, gate_w):
    return pl.pallas_call(
        _gate_body,
        grid=(T // GATE_TT,),
        in_specs=[
            pl.BlockSpec((GATE_TT, DIM), lambda t: (t, 0)),
            pl.BlockSpec((E, DIM), lambda t: (0, 0)),
        ],
        out_specs=[
            pl.BlockSpec((GATE_TT, SLOTS), lambda t: (t, 0)),
            pl.BlockSpec((GATE_TT, SLOTS), lambda t: (t, 0)),
        ],
        out_shape=[
            jax.ShapeDtypeStruct((T, SLOTS), jnp.int32),
            jax.ShapeDtypeStruct((T, SLOTS), jnp.float32),
        ],
    )(x, gate_w)


def _gmm1_body(sg_ref, used_ref, xs_ref, w1_ref, w3_ref, ws_ref, h_ref):
    t = pl.program_id(1)

    @pl.when(t < used_ref[0])
    def _():
        xb = xs_ref[...]
        a = jax.lax.dot_general(xb, w1_ref[0], (((1,), (1,)), ((), ())),
                                preferred_element_type=jnp.float32)
        b = jax.lax.dot_general(xb, w3_ref[0], (((1,), (1,)), ((), ())),
                                preferred_element_type=jnp.float32)
        h = (a * jax.nn.sigmoid(a)) * b
        h_ref[...] = h * ws_ref[...]


def _gmm2_body(sg_ref, used_ref, h_ref, w2_ref, o_ref):
    t = pl.program_id(1)

    @pl.when(t < used_ref[0])
    def _():
        hb = h_ref[...]
        o_ref[...] = jax.lax.dot_general(hb, w2_ref[0], (((1,), (1,)), ((), ())),
                                         preferred_element_type=jnp.float32)


def _gmm1(sg, used, xs, w1, w3, ws, n_split=2):
    nb = INTER // n_split

    def wmap(n, t, sg_ref, used_ref):
        return (jnp.minimum(sg_ref[t], E - 1), n, 0)

    def rowmap(n, t, sg_ref, used_ref):
        return (jnp.minimum(t, used_ref[0] - 1), 0)

    grid_spec = pltpu.PrefetchScalarGridSpec(
        num_scalar_prefetch=2,
        grid=(n_split, MAX_TILES),
        in_specs=[
            pl.BlockSpec((ROW_TILE, DIM), rowmap),
            pl.BlockSpec((1, nb, DIM), wmap),
            pl.BlockSpec((1, nb, DIM), wmap),
            pl.BlockSpec((ROW_TILE, 1), rowmap),
        ],
        out_specs=pl.BlockSpec(
            (ROW_TILE, nb),
            lambda n, t, sg_ref, used_ref: (jnp.minimum(t, used_ref[0] - 1), n)),
    )
    return pl.pallas_call(
        _gmm1_body,
        grid_spec=grid_spec,
        out_shape=jax.ShapeDtypeStruct((M_PAD, INTER), jnp.float32),
        compiler_params=pltpu.CompilerParams(
            dimension_semantics=("arbitrary", "arbitrary")),
    )(sg, used, xs, w1, w3, ws)


def _gmm2(sg, used, h, w2, n_split=2):
    nb = DIM // n_split

    def wmap(n, t, sg_ref, used_ref):
        return (jnp.minimum(sg_ref[t], E - 1), n, 0)

    def rowmap(n, t, sg_ref, used_ref):
        return (jnp.minimum(t, used_ref[0] - 1), 0)

    grid_spec = pltpu.PrefetchScalarGridSpec(
        num_scalar_prefetch=2,
        grid=(n_split, MAX_TILES),
        in_specs=[
            pl.BlockSpec((ROW_TILE, INTER), rowmap),
            pl.BlockSpec((1, nb, INTER), wmap),
        ],
        out_specs=pl.BlockSpec(
            (ROW_TILE, nb),
            lambda n, t, sg_ref, used_ref: (jnp.minimum(t, used_ref[0] - 1), n)),
    )
    return pl.pallas_call(
        _gmm2_body,
        grid_spec=grid_spec,
        out_shape=jax.ShapeDtypeStruct((M_PAD, DIM), jnp.float32),
        compiler_params=pltpu.CompilerParams(
            dimension_semantics=("arbitrary", "arbitrary")),
    )(sg, used, h, w2)


SH_TT = 256
SH_NB1 = 256
SH_NB2 = 1024
SH_INTER = N_SHARED * INTER


def _shared_up_body(x_ref, s1_ref, s3_ref, h_ref):
    xb = x_ref[...]
    a = jax.lax.dot_general(xb, s1_ref[...], (((1,), (1,)), ((), ())),
                            preferred_element_type=jnp.float32)
    b = jax.lax.dot_general(xb, s3_ref[...], (((1,), (1,)), ((), ())),
                            preferred_element_type=jnp.float32)
    h_ref[...] = (a * jax.nn.sigmoid(a)) * b


def _shared_up(x, s1, s3):
    return pl.pallas_call(
        _shared_up_body,
        grid=(SH_INTER // SH_NB1, T // SH_TT),
        in_specs=[
            pl.BlockSpec((SH_TT, DIM), lambda n, t: (t, 0)),
            pl.BlockSpec((SH_NB1, DIM), lambda n, t: (n, 0)),
            pl.BlockSpec((SH_NB1, DIM), lambda n, t: (n, 0)),
        ],
        out_specs=pl.BlockSpec((SH_TT, SH_NB1), lambda n, t: (t, n)),
        out_shape=jax.ShapeDtypeStruct((T, SH_INTER), jnp.float32),
        compiler_params=pltpu.CompilerParams(
            dimension_semantics=("arbitrary", "arbitrary")),
    )(x, s1, s3)


def _shared_down_body(h_ref, s2_ref, ym_ref, y_ref):
    hb = h_ref[...]
    o = jax.lax.dot_general(hb, s2_ref[...], (((1,), (1,)), ((), ())),
                            preferred_element_type=jnp.float32)
    y_ref[...] = o + ym_ref[...]


def _shared_down(h, s2, y_moe):
    return pl.pallas_call(
        _shared_down_body,
        grid=(DIM // SH_NB2, T // SH_TT),
        in_specs=[
            pl.BlockSpec((SH_TT, SH_INTER), lambda n, t: (t, 0)),
            pl.BlockSpec((SH_NB2, SH_INTER), lambda n, t: (n, 0)),
            pl.BlockSpec((SH_TT, SH_NB2), lambda n, t: (t, n)),
        ],
        out_specs=pl.BlockSpec((SH_TT, SH_NB2), lambda n, t: (t, n)),
        out_shape=jax.ShapeDtypeStruct((T, DIM), jnp.float32),
        compiler_params=pltpu.CompilerParams(
            dimension_semantics=("arbitrary", "arbitrary")),
    )(h, s2, y_moe)


def kernel(x, gate_w, w1, w3, w2, s1, s3, s2):
    idx8, w8 = _gate(x, gate_w)

    flat_e = idx8.reshape(-1)
    oh = (flat_e[:, None] == jnp.arange(E + 1)[None, :]).astype(jnp.int32)
    C = jnp.cumsum(oh, axis=0)
    counts = C[-1]
    rank = jnp.take_along_axis(C, flat_e[:, None], axis=1)[:, 0] - 1

    cnt = counts[:E]
    gtiles = (cnt + ROW_TILE - 1) // ROW_TILE
    tile_start = jnp.cumsum(gtiles) - gtiles
    pstart = tile_start * ROW_TILE
    used = jnp.sum(gtiles).astype(jnp.int32)

    pstart_full = jnp.concatenate([pstart, jnp.array([M_PAD], jnp.int32)])
    pos = pstart_full[flat_e] + rank
    pos_safe = jnp.minimum(pos, M_PAD)

    token_ids = jnp.zeros((M_PAD + 1,), jnp.int32).at[pos_safe].set(
        jnp.arange(FLAT, dtype=jnp.int32) // SLOTS, mode="drop")[:M_PAD]
    sorted_w = jnp.zeros((M_PAD + 1,), jnp.float32).at[pos_safe].set(
        w8.reshape(-1), mode="drop")[:M_PAD]

    step_group = jnp.repeat(jnp.arange(E, dtype=jnp.int32), gtiles,
                            total_repeat_length=MAX_TILES)
    used_arr = jnp.array([0], jnp.int32) + used

    sorted_x = jnp.take(x, token_ids, axis=0)
    h = _gmm1(step_group, used_arr, sorted_x, w1, w3, sorted_w[:, None])
    o = _gmm2(step_group, used_arr, h, w2)

    pos6 = pos.reshape(T, SLOTS)[:, :TOPK]
    y_moe = jnp.sum(jnp.take(o, pos6.reshape(-1), axis=0)
                    .reshape(T, TOPK, DIM), axis=1)

    hs = _shared_up(x, s1, s3)
    return _shared_down(hs, s2, y_moe)
